# Initial kernel scaffold; baseline (speedup 1.0000x reference)
import jax
import jax.numpy as jnp
from jax.experimental import pallas as pl


def kernel(input_ids, W0, W1, W2, W3, W4, W5, W6, W7):
    raise NotImplementedError("write your pallas kernel here")


# speedup vs baseline: 1.2307x; 1.2307x over previous
import functools

import jax
import jax.numpy as jnp
from jax import lax
from jax.experimental import pallas as pl
from jax.experimental.pallas import tpu as pltpu
from jax.experimental.pallas import tpu_sc as plsc

NUM_QUANT = 8
B, T, H = 4, 2048, 1024
NTOK = B * T
NC, NS, L = 2, 16, 16
NW = NC * NS
TPW = NTOK // NW
C = 8
NCHUNK = TPW // C
NSEG = H // L


def _body(ids_hbm, w0, w1, w2, w3, w4, w5, w6, w7,
          out_hbm, idx_v, buf, obuf, sem):
    tables = (w0, w1, w2, w3, w4, w5, w6, w7)
    wid = lax.axis_index("s") * NC + lax.axis_index("c")
    base = wid * TPW

    for i in range(NUM_QUANT):
        pltpu.sync_copy(ids_hbm.at[i, pl.ds(base, TPW)], idx_v.at[i])

    def chunk_body(c, carry):
        tok0 = c * C
        handles = []
        for i in range(NUM_QUANT):
            handles.append(pltpu.async_copy(
                tables[i].at[idx_v.at[i, pl.ds(tok0, C)]],
                buf.at[pl.ds(i * C, C)],
                sem,
            ))
        for h in handles:
            h.wait()

        def tok_body(t, carry_t):
            def seg_body(s, carry_s):
                col = pl.ds(s * L, L)
                acc = buf[t, col]
                for i in range(1, NUM_QUANT):
                    acc = acc + buf[i * C + t, col]
                obuf[t, col] = acc
                return carry_s
            return lax.fori_loop(0, NSEG, seg_body, carry_t)
        lax.fori_loop(0, C, tok_body, 0)

        pltpu.sync_copy(obuf, out_hbm.at[pl.ds(base + tok0, C)])
        return carry

    lax.fori_loop(0, NCHUNK, chunk_body, 0)


@functools.partial(
    pl.kernel,
    out_type=jax.ShapeDtypeStruct((NTOK, H), jnp.float32),
    mesh=plsc.VectorSubcoreMesh(core_axis_name="c", subcore_axis_name="s"),
    scratch_types=[
        pltpu.VMEM((NUM_QUANT, TPW), jnp.int32),
        pltpu.VMEM((NUM_QUANT * C, H), jnp.float32),
        pltpu.VMEM((C, H), jnp.float32),
        pltpu.SemaphoreType.DMA,
    ],
)
def _sc_kernel(*refs):
    _body(*refs)


def kernel(input_ids, W0, W1, W2, W3, W4, W5, W6, W7):
    ids = input_ids.reshape(NUM_QUANT, NTOK).astype(jnp.int32)
    out = _sc_kernel(ids, W0, W1, W2, W3, W4, W5, W6, W7)
    return out.reshape(B, T, H)


# speedup vs baseline: 1.4832x; 1.2052x over previous
import functools

import jax
import jax.numpy as jnp
from jax import lax
from jax.experimental import pallas as pl
from jax.experimental.pallas import tpu as pltpu
from jax.experimental.pallas import tpu_sc as plsc

NUM_QUANT = 8
B, T, H = 4, 2048, 1024
NTOK = B * T
NC, NS, L = 2, 16, 16
NW = NC * NS
TPW = NTOK // NW
C = 8
NCHUNK = TPW // C
NPAIR = NCHUNK // 2
NSEG = H // L
GC = 4 * C


def _body(ids_hbm, w0, w1, w2, w3, w4, w5, w6, w7,
          out_hbm, idx_v, gbuf0, gbuf1, acc0, acc1, sg0, sg1, so0, so1):
    tables = (w0, w1, w2, w3, w4, w5, w6, w7)
    wid = lax.axis_index("s") * NC + lax.axis_index("c")
    base = wid * TPW

    for i in range(NUM_QUANT):
        pltpu.sync_copy(ids_hbm.at[i, pl.ds(base, TPW)], idx_v.at[i])

    def fire_group(c, g, buf, sem):
        tok0 = pl.multiple_of(c * C, 8)
        for j in range(4):
            i = g * 4 + j
            pltpu.async_copy(
                tables[i].at[idx_v.at[i, pl.ds(tok0, C)]],
                buf.at[pl.ds(j * C, C)],
                sem,
            )

    def wait_gather(buf, sem):
        pltpu.make_async_copy(w0.at[pl.ds(0, GC)], buf, sem).wait()

    def wait_out(sem):
        pltpu.make_async_copy(acc0, out_hbm.at[pl.ds(0, C)], sem).wait()

    def compute_first(gbuf, acc):
        def tok(t, carry):
            for s in range(NSEG):
                col = pl.ds(s * L, L)
                acc[t, col] = ((gbuf[t, col] + gbuf[C + t, col])
                               + (gbuf[2 * C + t, col] + gbuf[3 * C + t, col]))
            return carry
        lax.fori_loop(0, C, tok, 0)

    def compute_second(gbuf, acc):
        def tok(t, carry):
            for s in range(NSEG):
                col = pl.ds(s * L, L)
                acc[t, col] = acc[t, col] + (
                    (gbuf[t, col] + gbuf[C + t, col])
                    + (gbuf[2 * C + t, col] + gbuf[3 * C + t, col]))
            return carry
        lax.fori_loop(0, C, tok, 0)

    def out_slot(c):
        return out_hbm.at[pl.ds(pl.multiple_of(base + c * C, 8), C)]

    fire_group(0, 0, gbuf0, sg0)

    def pair_body(cc, carry):
        c0 = 2 * cc
        c1 = c0 + 1
        fire_group(c0, 1, gbuf1, sg1)

        @pl.when(cc >= 1)
        def _wait_prev_out0():
            wait_out(so0)
        wait_gather(gbuf0, sg0)
        compute_first(gbuf0, acc0)
        fire_group(c1, 0, gbuf0, sg0)
        wait_gather(gbuf1, sg1)
        compute_second(gbuf1, acc0)
        pltpu.async_copy(acc0, out_slot(c0), so0)
        fire_group(c1, 1, gbuf1, sg1)

        @pl.when(cc >= 1)
        def _wait_prev_out1():
            wait_out(so1)
        wait_gather(gbuf0, sg0)
        compute_first(gbuf0, acc1)
        @pl.when(cc + 1 < NPAIR)
        def _fire_next_pair():
            fire_group(c1 + 1, 0, gbuf0, sg0)
        wait_gather(gbuf1, sg1)
        compute_second(gbuf1, acc1)
        pltpu.async_copy(acc1, out_slot(c1), so1)
        return carry

    lax.fori_loop(0, NPAIR, pair_body, 0)
    wait_out(so0)
    wait_out(so1)


@functools.partial(
    pl.kernel,
    out_type=jax.ShapeDtypeStruct((NTOK, H), jnp.float32),
    mesh=plsc.VectorSubcoreMesh(core_axis_name="c", subcore_axis_name="s"),
    scratch_types=[
        pltpu.VMEM((NUM_QUANT, TPW), jnp.int32),
        pltpu.VMEM((GC, H), jnp.float32),
        pltpu.VMEM((GC, H), jnp.float32),
        pltpu.VMEM((C, H), jnp.float32),
        pltpu.VMEM((C, H), jnp.float32),
        pltpu.SemaphoreType.DMA,
        pltpu.SemaphoreType.DMA,
        pltpu.SemaphoreType.DMA,
        pltpu.SemaphoreType.DMA,
    ],
)
def _sc_kernel(*refs):
    _body(*refs)


def kernel(input_ids, W0, W1, W2, W3, W4, W5, W6, W7):
    ids = input_ids.reshape(NUM_QUANT, NTOK).astype(jnp.int32)
    out = _sc_kernel(ids, W0, W1, W2, W3, W4, W5, W6, W7)
    return out.reshape(B, T, H)
